# baseline (speedup 1.0000x reference)
import functools

import jax
import jax.numpy as jnp
from jax import lax
from jax.experimental import pallas as pl
from jax.experimental.pallas import tpu as pltpu
from jax.experimental.pallas import tpu_sc as plsc

_N = 10000
_D = 128
_E = 320000
_B = 4096
_NC = 2
_NS = 16
_LL = 16
_CHUNK = 96
_CPT = 216
_CBLK = 24
_EPT = _CPT * _CHUNK
_EPAD = _EPT * _NS
_RPT = 640
_NPAD = _RPT * _NS
_NV = _D // _LL

_MESH = plsc.VectorSubcoreMesh(core_axis_name="c", subcore_axis_name="s")


_GDN = lax.GatherDimensionNumbers(
    offset_dims=(), collapsed_slice_dims=(0,), start_index_map=(0,))


def _bcast_lane(v16, e):
    idx = jnp.full((_LL, 1), e, jnp.int32)
    return lax.gather(v16, idx, _GDN, slice_sizes=(1,),
                      mode=lax.GatherScatterMode.PROMISE_IN_BOUNDS)


_BLK = 2000


def _mm_body(x_ref, w_ref, y_ref):
    y_ref[0] = jnp.dot(x_ref[0], w_ref[0], preferred_element_type=jnp.float32)


def _mm(x, w):
    return pl.pallas_call(
        _mm_body,
        grid=(_NC, _N // _BLK),
        in_specs=[pl.BlockSpec((1, _BLK, _D), lambda g, r: (g, r, 0)),
                  pl.BlockSpec((1, _D, _D), lambda g, r: (g, 0, 0))],
        out_specs=pl.BlockSpec((1, _BLK, _D), lambda g, r: (g, r, 0)),
        out_shape=jax.ShapeDtypeStruct((_NC, _N, _D), jnp.float32),
    )(x, w)


def _relu_mm_body(s_ref, w_ref, x_ref, y_ref):
    x = jnp.maximum(s_ref[0], 0.0)
    x_ref[0] = x
    y_ref[0] = jnp.dot(x, w_ref[0], preferred_element_type=jnp.float32)


def _relu_mm(s, w):
    return pl.pallas_call(
        _relu_mm_body,
        grid=(_NC, _N // _BLK),
        in_specs=[pl.BlockSpec((1, _BLK, _D), lambda g, r: (g, r, 0)),
                  pl.BlockSpec((1, _D, _D), lambda g, r: (g, 0, 0))],
        out_specs=[pl.BlockSpec((1, _BLK, _D), lambda g, r: (g, r, 0)),
                   pl.BlockSpec((1, _BLK, _D), lambda g, r: (g, r, 0))],
        out_shape=[jax.ShapeDtypeStruct((_NC, _N, _D), jnp.float32),
                   jax.ShapeDtypeStruct((_NC, _N, _D), jnp.float32)],
    )(s, w)


_DH = _D // 2
_NVH = _DH // _LL


@functools.partial(
    pl.kernel,
    out_type=jax.ShapeDtypeStruct((_NC, 2, _NPAD, _DH), jnp.float32),
    mesh=_MESH,
    scratch_types=[
        pltpu.VMEM((_CBLK, _CHUNK), jnp.int32),
        pltpu.VMEM((_CBLK, _CHUNK), jnp.int32),
        pltpu.VMEM((_CBLK, _CHUNK), jnp.float32),
        pltpu.VMEM((_CHUNK, _DH), jnp.float32),
        pltpu.VMEM((_CHUNK, _DH), jnp.float32),
        pltpu.VMEM((_CHUNK, _DH), jnp.float32),
        pltpu.VMEM_SHARED((_NPAD, _DH), jnp.float32),
        pltpu.VMEM_SHARED((_NPAD, _DH), jnp.float32),
        pltpu.SemaphoreType.DMA,
        pltpu.SemaphoreType.DMA,
        pltpu.SemaphoreType.DMA,
        pltpu.SemaphoreType.DMA,
        pltpu.SemaphoreType.DMA,
        pltpu.SemaphoreType.DMA,
    ],
    compiler_params=pltpu.CompilerParams(use_tc_tiling_on_sc=False),
)
def _spmm_pair_kernel(x_h, rows_h, cols_h, vals_h, out_h,
                      rows_v, cols_v, vals_v, g0, g1, g2, xs, acc,
                      gs0, gs1, gs2, ss0, ss1, ss2):
    c = lax.axis_index("c")
    s = lax.axis_index("s")
    gbufs = (g0, g1, g2)
    gsems = (gs0, gs1, gs2)
    ssems = (ss0, ss1, ss2)
    own = pl.ds(s * _RPT, _RPT)
    zero = jnp.zeros((_LL,), jnp.float32)

    for h in (0, 1):
        pltpu.sync_copy(x_h.at[c, h, own], xs.at[own])

        def zrow(i, carry):
            for v in range(_NVH):
                g0[i, pl.ds(v * _LL, _LL)] = zero
            return carry

        lax.fori_loop(0, _CHUNK, zrow, 0)
        for q in range(_RPT // _CHUNK):
            pltpu.sync_copy(g0, acc.at[pl.ds(s * _RPT + q * _CHUNK, _CHUNK)])
        _rem = _RPT - (_RPT // _CHUNK) * _CHUNK
        if _rem:
            pltpu.sync_copy(
                g0.at[pl.ds(0, _rem)],
                acc.at[pl.ds(s * _RPT + (_RPT // _CHUNK) * _CHUNK, _rem)])
        plsc.subcore_barrier()

        def block(bi, carry):
            sl_b = pl.ds(bi * _CBLK, _CBLK)
            pltpu.sync_copy(rows_h.at[c, s, sl_b], rows_v)
            pltpu.sync_copy(cols_h.at[c, s, sl_b], cols_v)
            pltpu.sync_copy(vals_h.at[c, s, sl_b], vals_v)

            pltpu.async_copy(xs.at[cols_v.at[0]], g0, gs0)
            pltpu.async_copy(xs.at[cols_v.at[1]], g1, gs1)

            def triple(j3, carry1):
                for b in (0, 1, 2):
                    j = j3 * 3 + b
                    gb = gbufs[b]
                    pltpu.make_async_copy(
                        xs.at[cols_v.at[j]], gb, gsems[b]).wait()

                    @pl.when(j + 2 < _CBLK)
                    def _():
                        ob = (b + 2) % 3

                        @pl.when(j >= 1)
                        def _():
                            pltpu.make_async_copy(
                                gbufs[ob], acc.at[rows_v.at[j - 1]],
                                ssems[ob]).wait()
                        pltpu.async_copy(xs.at[cols_v.at[j + 2]],
                                         gbufs[ob], gsems[ob])

                    def grp(g, carry2):
                        v16 = vals_v[j, pl.ds(g * _LL, _LL)]
                        for e in range(_LL):
                            row = g * _LL + e
                            vb = _bcast_lane(v16, e)
                            for v in range(_NVH):
                                sl = pl.ds(v * _LL, _LL)
                                gb[row, sl] = gb[row, sl] * vb
                        return carry2

                    lax.fori_loop(0, _CHUNK // _LL, grp, 0)
                    pltpu.async_copy(gb, acc.at[rows_v.at[j]], ssems[b],
                                     add=True)
                return carry1

            lax.fori_loop(0, _CBLK // 3, triple, 0)
            pltpu.make_async_copy(g0, acc.at[rows_v.at[_CBLK - 3]], ss0).wait()
            pltpu.make_async_copy(g1, acc.at[rows_v.at[_CBLK - 2]], ss1).wait()
            pltpu.make_async_copy(g2, acc.at[rows_v.at[_CBLK - 1]], ss2).wait()
            return carry

        lax.fori_loop(0, _CPT // _CBLK, block, 0)
        plsc.subcore_barrier()

        pltpu.sync_copy(acc.at[own], out_h.at[c, h, own])
        plsc.subcore_barrier()


_GC = (3 * _B) // (_NC * _NS * _CHUNK)


@functools.partial(
    pl.kernel,
    out_type=jax.ShapeDtypeStruct((_NC * _NS, _GC, _CHUNK, _D), jnp.float32),
    mesh=_MESH,
    scratch_types=[
        pltpu.VMEM((_GC, _CHUNK), jnp.int32),
        pltpu.VMEM((_GC, _CHUNK), jnp.float32),
        pltpu.VMEM((_GC, _CHUNK), jnp.float32),
        pltpu.VMEM((_CHUNK, _D), jnp.float32),
        pltpu.VMEM((_CHUNK, _D), jnp.float32),
        pltpu.SemaphoreType.DMA,
        pltpu.SemaphoreType.DMA,
    ],
)
def _gather_combine_kernel(x1_h, s2_h, idx_h, wa_h, wb_h, out_h,
                           idx_v, wa_v, wb_v, buf_a, buf_b, sem_a, sem_b):
    c = lax.axis_index("c")
    s = lax.axis_index("s")
    w = c * _NS + s
    pltpu.sync_copy(idx_h.at[w], idx_v)
    pltpu.sync_copy(wa_h.at[w], wa_v)
    pltpu.sync_copy(wb_h.at[w], wb_v)

    def chunk(j, carry):
        da = pltpu.async_copy(x1_h.at[idx_v.at[j]], buf_a, sem_a)
        db = pltpu.async_copy(s2_h.at[idx_v.at[j]], buf_b, sem_b)
        da.wait()
        db.wait()

        def grp(g, carry2):
            a16 = wa_v[j, pl.ds(g * _LL, _LL)]
            b16 = wb_v[j, pl.ds(g * _LL, _LL)]
            for e in range(_LL):
                r = g * _LL + e
                ab = _bcast_lane(a16, e)
                bb = _bcast_lane(b16, e)
                for v in range(_NV):
                    sl = pl.ds(v * _LL, _LL)
                    x2 = jnp.maximum(buf_b[r, sl], 0.0)
                    buf_a[r, sl] = jnp.maximum(ab * buf_a[r, sl] + bb * x2,
                                               0.0)
            return carry2

        lax.fori_loop(0, _CHUNK // _LL, grp, 0)
        pltpu.sync_copy(buf_a, out_h.at[w, j])
        return carry

    lax.fori_loop(0, _GC, chunk, 0)


def _to_halves(y):
    yp = jnp.pad(y, ((0, 0), (0, _NPAD - _N), (0, 0)))
    return yp.reshape(_NC, _NPAD, 2, _DH).transpose(0, 2, 1, 3)


def _from_halves(o):
    return o.transpose(0, 2, 1, 3).reshape(_NC, _NPAD, _D)


def _prep_edges(rows_u, cols_u, vals_u, rows_i, cols_i, vals_i):
    pad = _EPAD - _E

    def lay(a, fill):
        return jnp.pad(a, (0, pad), constant_values=fill).reshape(
            _NS, _CPT, _CHUNK)

    rows = jnp.stack([lay(rows_u, 0), lay(rows_i, 0)])
    cols = jnp.stack([lay(cols_u, 0), lay(cols_i, 0)])
    vals = jnp.stack([lay(vals_u, 0.0), lay(vals_i, 0.0)])
    return rows, cols, vals


def kernel(user_table, item_table, W_u0, W_u1, W_i0, W_i1, wb1, wb2,
           u0_rows, u0_cols, u0_vals, u1_rows, u1_cols, u1_vals,
           i0_rows, i0_cols, i0_vals, i1_rows, i1_cols, i1_vals,
           user_idx, item_idx, neg_item_idx):
    tables = jnp.stack([user_table, item_table])
    W0 = jnp.stack([W_u0, W_i0])
    W1 = jnp.stack([W_u1, W_i1])

    r1, c1, v1 = _prep_edges(u0_rows, u0_cols, u0_vals,
                             i0_rows, i0_cols, i0_vals)
    r2, c2, v2 = _prep_edges(u1_rows, u1_cols, u1_vals,
                             i1_rows, i1_cols, i1_vals)

    y0 = _mm(tables, W0)
    s1 = _from_halves(_spmm_pair_kernel(_to_halves(y0), r1, c1, v1))
    x1, y1 = _relu_mm(s1[:, :_N], W1)
    s2 = _from_halves(_spmm_pair_kernel(_to_halves(y1), r2, c2, v2))

    cat_idx = jnp.concatenate([user_idx, item_idx + _N, neg_item_idx + _N])
    wa1, wa2 = wb1[0, 0, 0], wb2[0, 0, 0]
    wc1, wc2 = wb1[1, 0, 0], wb2[1, 0, 0]
    wa = jnp.concatenate([jnp.full((_B,), wa1), jnp.full((_B,), wa2),
                          jnp.full((_B,), wa2)])
    wb = jnp.concatenate([jnp.full((_B,), wc1), jnp.full((_B,), wc2),
                          jnp.full((_B,), wc2)])
    shape = (_NC * _NS, _GC, _CHUNK)
    out = _gather_combine_kernel(
        x1.reshape(_NC * _N, _D), s2[:, :_N].reshape(_NC * _N, _D),
        cat_idx.reshape(shape).astype(jnp.int32),
        wa.reshape(shape), wb.reshape(shape))
    flat = out.reshape(3 * _B, _D)
    return flat[:_B], flat[_B:2 * _B], flat[2 * _B:]


# speedup vs baseline: 1.4621x; 1.4621x over previous
import functools

import jax
import jax.numpy as jnp
from jax import lax
from jax.experimental import pallas as pl
from jax.experimental.pallas import tpu as pltpu
from jax.experimental.pallas import tpu_sc as plsc

_N = 10000
_D = 128
_E = 320000
_B = 4096
_NC = 2
_NS = 16
_LL = 16
_CHUNK = 128
_CPT = 160
_CBLK = 32
_EPT = _CPT * _CHUNK
_EPAD = _EPT * _NS
_RPT = 640
_NPAD = _RPT * _NS
_NV = _D // _LL

_MESH = plsc.VectorSubcoreMesh(core_axis_name="c", subcore_axis_name="s")


_GDN = lax.GatherDimensionNumbers(
    offset_dims=(), collapsed_slice_dims=(0,), start_index_map=(0,))


def _bcast_lane(v16, e):
    idx = jnp.full((_LL, 1), e, jnp.int32)
    return lax.gather(v16, idx, _GDN, slice_sizes=(1,),
                      mode=lax.GatherScatterMode.PROMISE_IN_BOUNDS)


_BLK = 2000


def _mm_body(x_ref, w_ref, y_ref):
    y_ref[0] = jnp.dot(x_ref[0], w_ref[0], preferred_element_type=jnp.float32)


def _mm(x, w):
    return pl.pallas_call(
        _mm_body,
        grid=(_NC, _N // _BLK),
        in_specs=[pl.BlockSpec((1, _BLK, _D), lambda g, r: (g, r, 0)),
                  pl.BlockSpec((1, _D, _D), lambda g, r: (g, 0, 0))],
        out_specs=pl.BlockSpec((1, _BLK, _D), lambda g, r: (g, r, 0)),
        out_shape=jax.ShapeDtypeStruct((_NC, _N, _D), jnp.float32),
    )(x, w)


def _relu_mm_body(s_ref, w_ref, x_ref, y_ref):
    x = jnp.maximum(s_ref[0], 0.0)
    x_ref[0] = x
    y_ref[0] = jnp.dot(x, w_ref[0], preferred_element_type=jnp.float32)


def _relu_mm(s, w):
    return pl.pallas_call(
        _relu_mm_body,
        grid=(_NC, _N // _BLK),
        in_specs=[pl.BlockSpec((1, _BLK, _D), lambda g, r: (g, r, 0)),
                  pl.BlockSpec((1, _D, _D), lambda g, r: (g, 0, 0))],
        out_specs=[pl.BlockSpec((1, _BLK, _D), lambda g, r: (g, r, 0)),
                   pl.BlockSpec((1, _BLK, _D), lambda g, r: (g, r, 0))],
        out_shape=[jax.ShapeDtypeStruct((_NC, _N, _D), jnp.float32),
                   jax.ShapeDtypeStruct((_NC, _N, _D), jnp.float32)],
    )(s, w)


@functools.partial(
    pl.kernel,
    out_type=jax.ShapeDtypeStruct((_NC, _NPAD, _D), jnp.float32),
    mesh=_MESH,
    scratch_types=[
        pltpu.VMEM((_CBLK, _CHUNK), jnp.int32),
        pltpu.VMEM((_CBLK, _CHUNK), jnp.int32),
        pltpu.VMEM((_CBLK, _CHUNK), jnp.float32),
        pltpu.VMEM((_CHUNK, _D), jnp.float32),
        pltpu.VMEM((_CHUNK, _D), jnp.float32),
        pltpu.VMEM_SHARED((_NPAD, _D), jnp.float32),
        pltpu.SemaphoreType.DMA,
        pltpu.SemaphoreType.DMA,
        pltpu.SemaphoreType.DMA,
        pltpu.SemaphoreType.DMA,
    ],
)
def _spmm_pair_kernel(x_h, rows_h, cols_h, vals_h, out_h,
                      rows_v, cols_v, vals_v, g0, g1, acc,
                      gs0, gs1, ss0, ss1):
    c = lax.axis_index("c")
    s = lax.axis_index("s")
    gbufs, gsems, ssems = (g0, g1), (gs0, gs1), (ss0, ss1)

    zero = jnp.zeros((_LL,), jnp.float32)

    def zrow(i, carry):
        for v in range(_NV):
            g0[i, pl.ds(v * _LL, _LL)] = zero
        return carry

    lax.fori_loop(0, _CHUNK, zrow, 0)
    for k5 in range(_RPT // _CHUNK):
        pltpu.sync_copy(g0, acc.at[pl.ds(s * _RPT + k5 * _CHUNK, _CHUNK)])
    plsc.subcore_barrier()

    def block(bi, carry):
        sl_b = pl.ds(bi * _CBLK, _CBLK)
        pltpu.sync_copy(rows_h.at[c, s, sl_b], rows_v)
        pltpu.sync_copy(cols_h.at[c, s, sl_b], cols_v)
        pltpu.sync_copy(vals_h.at[c, s, sl_b], vals_v)

        pltpu.async_copy(x_h.at[cols_v.at[0]], g0, gs0)

        def pair(j2, carry1):
            for b in (0, 1):
                j = j2 * 2 + b
                gb, gsem = gbufs[b], gsems[b]
                ob, osem, ossem = gbufs[1 - b], gsems[1 - b], ssems[1 - b]
                pltpu.make_async_copy(x_h.at[cols_v.at[j]], gb, gsem).wait()

                @pl.when(j >= 1)
                def _():
                    pltpu.make_async_copy(
                        ob, acc.at[rows_v.at[j - 1]], ossem).wait()

                @pl.when(j + 1 < _CBLK)
                def _():
                    pltpu.async_copy(x_h.at[cols_v.at[j + 1]], ob, osem)

                def grp(g, carry2):
                    v16 = vals_v[j, pl.ds(g * _LL, _LL)]
                    for e in range(_LL):
                        row = g * _LL + e
                        vb = _bcast_lane(v16, e)
                        for v in range(_NV):
                            sl = pl.ds(v * _LL, _LL)
                            gb[row, sl] = gb[row, sl] * vb
                    return carry2

                lax.fori_loop(0, _CHUNK // _LL, grp, 0)
                pltpu.async_copy(gb, acc.at[rows_v.at[j]], ssems[b], add=True)
            return carry1

        lax.fori_loop(0, _CBLK // 2, pair, 0)
        pltpu.make_async_copy(g1, acc.at[rows_v.at[_CBLK - 1]], ss1).wait()
        return carry

    lax.fori_loop(0, _CPT // _CBLK, block, 0)
    plsc.subcore_barrier()

    pltpu.sync_copy(acc.at[pl.ds(s * _RPT, _RPT)],
                    out_h.at[c, pl.ds(s * _RPT, _RPT)])


_GC = (3 * _B) // (_NC * _NS * _CHUNK)


@functools.partial(
    pl.kernel,
    out_type=jax.ShapeDtypeStruct((_NC * _NS, _GC, _CHUNK, _D), jnp.float32),
    mesh=_MESH,
    scratch_types=[
        pltpu.VMEM((_GC, _CHUNK), jnp.int32),
        pltpu.VMEM((_GC, _CHUNK), jnp.float32),
        pltpu.VMEM((_GC, _CHUNK), jnp.float32),
        pltpu.VMEM((_CHUNK, _D), jnp.float32),
        pltpu.VMEM((_CHUNK, _D), jnp.float32),
        pltpu.SemaphoreType.DMA,
        pltpu.SemaphoreType.DMA,
    ],
)
def _gather_combine_kernel(x1_h, s2_h, idx_h, wa_h, wb_h, out_h,
                           idx_v, wa_v, wb_v, buf_a, buf_b, sem_a, sem_b):
    c = lax.axis_index("c")
    s = lax.axis_index("s")
    w = c * _NS + s
    pltpu.sync_copy(idx_h.at[w], idx_v)
    pltpu.sync_copy(wa_h.at[w], wa_v)
    pltpu.sync_copy(wb_h.at[w], wb_v)

    def chunk(j, carry):
        da = pltpu.async_copy(x1_h.at[idx_v.at[j]], buf_a, sem_a)
        db = pltpu.async_copy(s2_h.at[idx_v.at[j]], buf_b, sem_b)
        da.wait()
        db.wait()

        def grp(g, carry2):
            a16 = wa_v[j, pl.ds(g * _LL, _LL)]
            b16 = wb_v[j, pl.ds(g * _LL, _LL)]
            for e in range(_LL):
                r = g * _LL + e
                ab = _bcast_lane(a16, e)
                bb = _bcast_lane(b16, e)
                for v in range(_NV):
                    sl = pl.ds(v * _LL, _LL)
                    x2 = jnp.maximum(buf_b[r, sl], 0.0)
                    buf_a[r, sl] = jnp.maximum(ab * buf_a[r, sl] + bb * x2,
                                               0.0)
            return carry2

        lax.fori_loop(0, _CHUNK // _LL, grp, 0)
        pltpu.sync_copy(buf_a, out_h.at[w, j])
        return carry

    lax.fori_loop(0, _GC, chunk, 0)


def _prep_edges(rows_u, cols_u, vals_u, rows_i, cols_i, vals_i):
    pad = _EPAD - _E

    def lay(a, fill):
        return jnp.pad(a, (0, pad), constant_values=fill).reshape(
            _NS, _CPT, _CHUNK)

    rows = jnp.stack([lay(rows_u, 0), lay(rows_i, 0)])
    cols = jnp.stack([lay(cols_u, 0), lay(cols_i + _N, _N)])
    vals = jnp.stack([lay(vals_u, 0.0), lay(vals_i, 0.0)])
    return rows, cols, vals


def kernel(user_table, item_table, W_u0, W_u1, W_i0, W_i1, wb1, wb2,
           u0_rows, u0_cols, u0_vals, u1_rows, u1_cols, u1_vals,
           i0_rows, i0_cols, i0_vals, i1_rows, i1_cols, i1_vals,
           user_idx, item_idx, neg_item_idx):
    tables = jnp.stack([user_table, item_table])
    W0 = jnp.stack([W_u0, W_i0])
    W1 = jnp.stack([W_u1, W_i1])

    r1, c1, v1 = _prep_edges(u0_rows, u0_cols, u0_vals,
                             i0_rows, i0_cols, i0_vals)
    r2, c2, v2 = _prep_edges(u1_rows, u1_cols, u1_vals,
                             i1_rows, i1_cols, i1_vals)

    y0 = _mm(tables, W0)
    s1 = _spmm_pair_kernel(y0.reshape(_NC * _N, _D), r1, c1, v1)
    x1, y1 = _relu_mm(s1[:, :_N], W1)
    s2 = _spmm_pair_kernel(y1.reshape(_NC * _N, _D), r2, c2, v2)

    cat_idx = jnp.concatenate([user_idx, item_idx + _N, neg_item_idx + _N])
    wa1, wa2 = wb1[0, 0, 0], wb2[0, 0, 0]
    wc1, wc2 = wb1[1, 0, 0], wb2[1, 0, 0]
    wa = jnp.concatenate([jnp.full((_B,), wa1), jnp.full((_B,), wa2),
                          jnp.full((_B,), wa2)])
    wb = jnp.concatenate([jnp.full((_B,), wc1), jnp.full((_B,), wc2),
                          jnp.full((_B,), wc2)])
    shape = (_NC * _NS, _GC, _CHUNK)
    out = _gather_combine_kernel(
        x1.reshape(_NC * _N, _D), s2[:, :_N].reshape(_NC * _N, _D),
        cat_idx.reshape(shape).astype(jnp.int32),
        wa.reshape(shape), wb.reshape(shape))
    flat = out.reshape(3 * _B, _D)
    return flat[:_B], flat[_B:2 * _B], flat[2 * _B:]
